# baseline (speedup 1.0000x reference)
import functools

import jax
import jax.numpy as jnp
from jax import lax
from jax.experimental import pallas as pl
from jax.experimental.pallas import tpu as pltpu
from jax.experimental.pallas import tpu_sc as plsc

B, S, D, E, TOP_K = 4, 2048, 1024, 8, 2
DFF = 4 * D
N = B * S
P = N * TOP_K
T = 512
R = P + E * T
NT = R // T
FT = 1024

NC, NS = 2, 16
NW = NC * NS
TPW = N // NW


_GT = 512


def _gating_body(x_ref, w_ref, b_ref, wts_ref, idx_ref, xi_ref):
    x = x_ref[...]
    g = jnp.dot(x, w_ref[...], preferred_element_type=jnp.float32)
    g = (g + b_ref[...]) / (1.0 + 1e-06)
    iota = lax.broadcasted_iota(jnp.int32, g.shape, 1)
    m1 = jnp.max(g, axis=1, keepdims=True)
    a1 = jnp.min(jnp.where(g == m1, iota, E), axis=1, keepdims=True)
    g2 = jnp.where(iota == a1, -jnp.inf, g)
    m2 = jnp.max(g2, axis=1, keepdims=True)
    a2 = jnp.min(jnp.where(g2 == m2, iota, E), axis=1, keepdims=True)
    e = jnp.exp(m2 - m1)
    s = 1.0 + e
    wts_ref[...] = jnp.concatenate([1.0 / s, e / s], axis=1)
    idx_ref[...] = jnp.concatenate([a1, a2], axis=1)
    xb = x.astype(jnp.bfloat16)
    lo = lax.bitcast_convert_type(xb[:, : D // 2], jnp.uint16).astype(jnp.uint32)
    hi = lax.bitcast_convert_type(xb[:, D // 2 :], jnp.uint16).astype(jnp.uint32)
    xi_ref[...] = lax.bitcast_convert_type(lo | (hi << 16), jnp.int32)


def _gating(xf, gate_W, gate_b):
    return pl.pallas_call(
        _gating_body,
        grid=(N // _GT,),
        in_specs=[
            pl.BlockSpec((_GT, D), lambda i: (i, 0)),
            pl.BlockSpec((D, E), lambda i: (0, 0)),
            pl.BlockSpec((1, E), lambda i: (0, 0)),
        ],
        out_specs=[
            pl.BlockSpec((_GT, TOP_K), lambda i: (i, 0)),
            pl.BlockSpec((_GT, TOP_K), lambda i: (i, 0)),
            pl.BlockSpec((_GT, D // 2), lambda i: (i, 0)),
        ],
        out_shape=[
            jax.ShapeDtypeStruct((N, TOP_K), jnp.float32),
            jax.ShapeDtypeStruct((N, TOP_K), jnp.int32),
            jax.ShapeDtypeStruct((N, D // 2), jnp.int32),
        ],
    )(xf, gate_W, gate_b.reshape(1, E))


_MG = 128
_NG = N // _MG


def _route_meta(idx):
    eids = jnp.arange(E, dtype=jnp.int32)
    A = (idx[:, 0][:, None] == eids).astype(jnp.float32)
    Bo = (idx[:, 1][:, None] == eids).astype(jnp.float32)
    tril_g = jnp.tril(jnp.ones((_NG, _NG), jnp.float32), k=-1)
    tril_m = jnp.tril(jnp.ones((_MG, _MG), jnp.float32))

    def excl_prefix(X):
        X3 = X.reshape(_NG, _MG, E)
        s = X3.sum(axis=1)
        gex = jnp.einsum("ij,je->ie", tril_g, s)
        incl = jnp.einsum("lk,gke->gle", tril_m, X3)
        return (incl - X3 + gex[:, None, :]).reshape(N, E), s.sum(axis=0)

    exA, cA = excl_prefix(A)
    exB, cB = excl_prefix(Bo)
    counts = (cA + cB).astype(jnp.int32)
    pc = ((counts + T - 1) // T) * T
    offs = jnp.concatenate([jnp.zeros(1, jnp.int32), jnp.cumsum(pc)]).astype(jnp.int32)
    rank0 = jnp.sum(A * (exA + exB), axis=1)
    rank1 = jnp.sum(Bo * (exA + A + exB), axis=1)
    offs_f = offs[:E].astype(jnp.float32)
    d0 = (rank0 + jnp.sum(A * offs_f, axis=1)).astype(jnp.int32)
    d1 = (rank1 + jnp.sum(Bo * offs_f, axis=1)).astype(jnp.int32)
    tile_starts = jnp.arange(NT, dtype=jnp.int32) * T
    tile_expert = jnp.clip(
        jnp.sum((tile_starts[:, None] >= offs[None, 1:E]).astype(jnp.int32), axis=1),
        0, E - 1,
    ).astype(jnp.int32)
    ntiles = (offs[E] // T).reshape(1)
    return tile_expert, ntiles, d0, d1


def _sc_mesh():
    return plsc.VectorSubcoreMesh(core_axis_name="c", subcore_axis_name="s")


_DCH = 64


def _dispatch(xi, d0, d1):
    nch = TPW // _DCH

    @functools.partial(
        pl.kernel,
        out_type=jax.ShapeDtypeStruct((R, D // 2), jnp.int32),
        mesh=_sc_mesh(),
        scratch_types=[
            pltpu.VMEM((_DCH,), jnp.int32), pltpu.VMEM((_DCH,), jnp.int32),
            pltpu.VMEM((_DCH,), jnp.int32), pltpu.VMEM((_DCH,), jnp.int32),
            pltpu.VMEM((_DCH, D // 2), jnp.int32), pltpu.VMEM((_DCH, D // 2), jnp.int32),
            pltpu.SemaphoreType.DMA, pltpu.SemaphoreType.DMA,
            pltpu.SemaphoreType.DMA, pltpu.SemaphoreType.DMA,
            pltpu.SemaphoreType.DMA, pltpu.SemaphoreType.DMA,
        ],
    )
    def dispatch_kernel(x_hbm, d0_hbm, d1_hbm, xs_hbm,
                        i0a, i0b, i1a, i1b, ra, rb,
                        la, lb, s0a, s0b, s1a, s1b):
        wid = lax.axis_index("s") * NC + lax.axis_index("c")
        idx0 = [i0a, i0b]
        idx1 = [i1a, i1b]
        rows = [ra, rb]
        lsem = [la, lb]
        ssem0 = [s0a, s0b]
        ssem1 = [s1a, s1b]
        scat = [[None, None], [None, None]]
        for j in range(nch):
            b = j % 2
            base = wid * TPW + j * _DCH
            if scat[b][0] is not None:
                scat[b][0].wait()
                scat[b][1].wait()
            load = pltpu.async_copy(x_hbm.at[pl.ds(base, _DCH)], rows[b], lsem[b])
            pltpu.sync_copy(d0_hbm.at[pl.ds(base, _DCH)], idx0[b])
            pltpu.sync_copy(d1_hbm.at[pl.ds(base, _DCH)], idx1[b])
            load.wait()
            scat[b][0] = pltpu.async_copy(rows[b], xs_hbm.at[idx0[b]], ssem0[b])
            scat[b][1] = pltpu.async_copy(rows[b], xs_hbm.at[idx1[b]], ssem1[b])
        for b in range(2):
            if scat[b][0] is not None:
                scat[b][0].wait()
                scat[b][1].wait()

    return dispatch_kernel(xi, d0, d1)


def _ffn_body(te_ref, nt_ref, xs_ref, w1_ref, b1_ref, w2_ref, b2_ref, y_ref):
    i = pl.program_id(0)

    @pl.when(i < nt_ref[0])
    def _():
        xi = xs_ref[...]
        xlo = lax.bitcast_convert_type(
            (xi & 0xFFFF).astype(jnp.uint16), jnp.bfloat16
        )
        xhi = lax.bitcast_convert_type(
            lax.shift_right_logical(xi, 16).astype(jnp.uint16), jnp.bfloat16
        )
        h = jnp.dot(
            xlo, w1_ref[0, pl.ds(0, D // 2), :], preferred_element_type=jnp.float32
        ) + jnp.dot(
            xhi, w1_ref[0, pl.ds(D // 2, D // 2), :], preferred_element_type=jnp.float32
        )
        h = h + b1_ref[0, 0]
        h = h * (1.0 / (1.0 + jnp.exp(-h)))
        y_ref[...] = (
            jnp.dot(h.astype(jnp.bfloat16), w2_ref[0], preferred_element_type=jnp.float32)
            + b2_ref[0]
        )


def _ffn(xs, W1, b1, W2, b2, tile_expert, ntiles):
    grid_spec = pltpu.PrefetchScalarGridSpec(
        num_scalar_prefetch=2,
        grid=(NT,),
        in_specs=[
            pl.BlockSpec((T, D // 2), lambda i, te, nt: (i, 0)),
            pl.BlockSpec((1, D, DFF), lambda i, te, nt: (te[i], 0, 0)),
            pl.BlockSpec((1, 1, DFF), lambda i, te, nt: (te[i], 0, 0)),
            pl.BlockSpec((1, DFF, D), lambda i, te, nt: (te[i], 0, 0)),
            pl.BlockSpec((1, 1, D), lambda i, te, nt: (te[i], 0, 0)),
        ],
        out_specs=pl.BlockSpec((T, D), lambda i, te, nt: (i, 0)),
    )
    return pl.pallas_call(
        _ffn_body,
        grid_spec=grid_spec,
        out_shape=jax.ShapeDtypeStruct((R, D), jnp.float32),
        compiler_params=pltpu.CompilerParams(
            dimension_semantics=("arbitrary",),
            vmem_limit_bytes=128 * 1024 * 1024,
        ),
    )(tile_expert, ntiles, xs, W1.astype(jnp.bfloat16), b1.reshape(E, 1, DFF),
      W2.astype(jnp.bfloat16), b2.reshape(E, 1, D))


_CCH = 32


def _combine(y, d0, d1):
    units = []
    for j in range(TPW // _CCH):
        units.append((j, 0))
        units.append((j, 1))

    @functools.partial(
        pl.kernel,
        out_type=[
            jax.ShapeDtypeStruct((N, D), jnp.float32),
            jax.ShapeDtypeStruct((N, D), jnp.float32),
        ],
        mesh=_sc_mesh(),
        scratch_types=[
            pltpu.VMEM((_CCH,), jnp.int32), pltpu.VMEM((_CCH,), jnp.int32),
            pltpu.VMEM((_CCH, D), jnp.float32), pltpu.VMEM((_CCH, D), jnp.float32),
            pltpu.SemaphoreType.DMA, pltpu.SemaphoreType.DMA,
            pltpu.SemaphoreType.DMA, pltpu.SemaphoreType.DMA,
        ],
    )
    def combine_kernel(y_hbm, d0_hbm, d1_hbm, g0_hbm, g1_hbm,
                       ia, ib, ra, rb, ga, gb, wa, wb):
        wid = lax.axis_index("s") * NC + lax.axis_index("c")
        idx = [ia, ib]
        rows = [ra, rb]
        gsem = [ga, gb]
        wsem = [wa, wb]
        g = [None, None]
        w = [None, None]

        def out_ref(k):
            return g0_hbm if k == 0 else g1_hbm

        for u, (j, k) in enumerate(units):
            b = u % 2
            base = wid * TPW + j * _CCH
            if w[b] is not None:
                w[b].wait()
            d_hbm = d0_hbm if k == 0 else d1_hbm
            pltpu.sync_copy(d_hbm.at[pl.ds(base, _CCH)], idx[b])
            g[b] = pltpu.async_copy(y_hbm.at[idx[b]], rows[b], gsem[b])
            if u >= 1:
                pb = (u - 1) % 2
                pj, pk = units[u - 1]
                pbase = wid * TPW + pj * _CCH
                g[pb].wait()
                w[pb] = pltpu.async_copy(
                    rows[pb], out_ref(pk).at[pl.ds(pbase, _CCH)], wsem[pb]
                )
        ub = (len(units) - 1) % 2
        lj, lk = units[-1]
        g[ub].wait()
        wlast = pltpu.async_copy(
            rows[ub], out_ref(lk).at[pl.ds(wid * TPW + lj * _CCH, _CCH)], wsem[ub]
        )
        if w[1 - ub] is not None:
            w[1 - ub].wait()
        wlast.wait()

    return combine_kernel(y, d0, d1)


_AT = 1024


def _add_body(a_ref, b_ref, w0_ref, w1_ref, o_ref):
    o_ref[...] = a_ref[...] * w0_ref[...] + b_ref[...] * w1_ref[...]


def _final_add(g0, g1, w0, w1):
    return pl.pallas_call(
        _add_body,
        grid=(N // _AT,),
        in_specs=[
            pl.BlockSpec((_AT, D), lambda i: (i, 0)),
            pl.BlockSpec((_AT, D), lambda i: (i, 0)),
            pl.BlockSpec((_AT, 1), lambda i: (i, 0)),
            pl.BlockSpec((_AT, 1), lambda i: (i, 0)),
        ],
        out_specs=pl.BlockSpec((_AT, D), lambda i: (i, 0)),
        out_shape=jax.ShapeDtypeStruct((N, D), jnp.float32),
    )(g0, g1, w0, w1)


def kernel(x, gate_W, gate_b, W1, b1, W2, b2):
    xf = x.reshape(N, D)
    wts, idx, xi = _gating(xf, gate_W, gate_b)
    tile_expert, ntiles, d0, d1 = _route_meta(idx)
    xs = _dispatch(xi, d0, d1)
    y = _ffn(xs, W1, b1, W2, b2, tile_expert, ntiles)
    g0, g1 = _combine(y, d0, d1)
    out = _final_add(g0, g1, wts[:, 0:1], wts[:, 1:2])
    return out.reshape(B, S, D)


# speedup vs baseline: 1.0442x; 1.0442x over previous
import functools

import jax
import jax.numpy as jnp
from jax import lax
from jax.experimental import pallas as pl
from jax.experimental.pallas import tpu as pltpu
from jax.experimental.pallas import tpu_sc as plsc

B, S, D, E, TOP_K = 4, 2048, 1024, 8, 2
DFF = 4 * D
N = B * S
P = N * TOP_K
T = 512
R = P + E * T
NT = R // T
FT = 1024

NC, NS = 2, 16
NW = NC * NS
TPW = N // NW


_GT = 512


def _gating_body(x_ref, w_ref, b_ref, w0_ref, w1_ref, idx_ref):
    x = x_ref[...]
    g = jnp.dot(x, w_ref[...], preferred_element_type=jnp.float32)
    g = (g + b_ref[...]) / (1.0 + 1e-06)
    iota = lax.broadcasted_iota(jnp.int32, g.shape, 1)
    m1 = jnp.max(g, axis=1, keepdims=True)
    a1 = jnp.min(jnp.where(g == m1, iota, E), axis=1, keepdims=True)
    g2 = jnp.where(iota == a1, -jnp.inf, g)
    m2 = jnp.max(g2, axis=1, keepdims=True)
    a2 = jnp.min(jnp.where(g2 == m2, iota, E), axis=1, keepdims=True)
    e = jnp.exp(m2 - m1)
    s = 1.0 + e
    w0_ref[...] = jnp.broadcast_to(1.0 / s, (_GT, 16))
    w1_ref[...] = jnp.broadcast_to(e / s, (_GT, 16))
    idx_ref[...] = jnp.concatenate([a1, a2], axis=1)


def _gating(xf, gate_W, gate_b):
    return pl.pallas_call(
        _gating_body,
        grid=(N // _GT,),
        in_specs=[
            pl.BlockSpec((_GT, D), lambda i: (i, 0)),
            pl.BlockSpec((D, E), lambda i: (0, 0)),
            pl.BlockSpec((1, E), lambda i: (0, 0)),
        ],
        out_specs=[
            pl.BlockSpec((_GT, 16), lambda i: (i, 0)),
            pl.BlockSpec((_GT, 16), lambda i: (i, 0)),
            pl.BlockSpec((_GT, TOP_K), lambda i: (i, 0)),
        ],
        out_shape=[
            jax.ShapeDtypeStruct((N, 16), jnp.float32),
            jax.ShapeDtypeStruct((N, 16), jnp.float32),
            jax.ShapeDtypeStruct((N, TOP_K), jnp.int32),
        ],
    )(xf, gate_W, gate_b.reshape(1, E))


_MG = 128
_NG = N // _MG


def _route_meta(idx):
    eids = jnp.arange(E, dtype=jnp.int32)
    A = (idx[:, 0][:, None] == eids).astype(jnp.float32)
    Bo = (idx[:, 1][:, None] == eids).astype(jnp.float32)
    tril_g = jnp.tril(jnp.ones((_NG, _NG), jnp.float32), k=-1)
    tril_m = jnp.tril(jnp.ones((_MG, _MG), jnp.float32))

    def excl_prefix(X):
        X3 = X.reshape(_NG, _MG, E)
        s = X3.sum(axis=1)
        gex = jnp.einsum("ij,je->ie", tril_g, s)
        incl = jnp.einsum("lk,gke->gle", tril_m, X3)
        return (incl - X3 + gex[:, None, :]).reshape(N, E), s.sum(axis=0)

    exA, cA = excl_prefix(A)
    exB, cB = excl_prefix(Bo)
    counts = (cA + cB).astype(jnp.int32)
    pc = ((counts + T - 1) // T) * T
    offs = jnp.concatenate([jnp.zeros(1, jnp.int32), jnp.cumsum(pc)]).astype(jnp.int32)
    rank0 = jnp.sum(A * (exA + exB), axis=1)
    rank1 = jnp.sum(Bo * (exA + A + exB), axis=1)
    offs_f = offs[:E].astype(jnp.float32)
    d0 = (rank0 + jnp.sum(A * offs_f, axis=1)).astype(jnp.int32)
    d1 = (rank1 + jnp.sum(Bo * offs_f, axis=1)).astype(jnp.int32)
    tile_starts = jnp.arange(NT, dtype=jnp.int32) * T
    tile_expert = jnp.clip(
        jnp.sum((tile_starts[:, None] >= offs[None, 1:E]).astype(jnp.int32), axis=1),
        0, E - 1,
    ).astype(jnp.int32)
    ntiles = (offs[E] // T).reshape(1)
    return tile_expert, ntiles, d0, d1


def _sc_mesh():
    return plsc.VectorSubcoreMesh(core_axis_name="c", subcore_axis_name="s")


_DCH = 32


def _dispatch(xf, d0, d1):
    nch = TPW // _DCH

    @functools.partial(
        pl.kernel,
        out_type=jax.ShapeDtypeStruct((R, D), jnp.float32),
        mesh=_sc_mesh(),
        scratch_types=[
            pltpu.VMEM((_DCH,), jnp.int32), pltpu.VMEM((_DCH,), jnp.int32),
            pltpu.VMEM((_DCH,), jnp.int32), pltpu.VMEM((_DCH,), jnp.int32),
            pltpu.VMEM((_DCH, D), jnp.float32), pltpu.VMEM((_DCH, D), jnp.float32),
            pltpu.SemaphoreType.DMA, pltpu.SemaphoreType.DMA,
            pltpu.SemaphoreType.DMA, pltpu.SemaphoreType.DMA,
            pltpu.SemaphoreType.DMA, pltpu.SemaphoreType.DMA,
        ],
    )
    def dispatch_kernel(x_hbm, d0_hbm, d1_hbm, xs_hbm,
                        i0a, i0b, i1a, i1b, ra, rb,
                        la, lb, s0a, s0b, s1a, s1b):
        wid = lax.axis_index("s") * NC + lax.axis_index("c")
        idx0 = [i0a, i0b]
        idx1 = [i1a, i1b]
        rows = [ra, rb]
        lsem = [la, lb]
        ssem0 = [s0a, s0b]
        ssem1 = [s1a, s1b]
        scat = [[None, None], [None, None]]
        for j in range(nch):
            b = j % 2
            base = wid * TPW + j * _DCH
            if scat[b][0] is not None:
                scat[b][0].wait()
                scat[b][1].wait()
            load = pltpu.async_copy(x_hbm.at[pl.ds(base, _DCH)], rows[b], lsem[b])
            pltpu.sync_copy(d0_hbm.at[pl.ds(base, _DCH)], idx0[b])
            pltpu.sync_copy(d1_hbm.at[pl.ds(base, _DCH)], idx1[b])
            load.wait()
            scat[b][0] = pltpu.async_copy(rows[b], xs_hbm.at[idx0[b]], ssem0[b])
            scat[b][1] = pltpu.async_copy(rows[b], xs_hbm.at[idx1[b]], ssem1[b])
        for b in range(2):
            if scat[b][0] is not None:
                scat[b][0].wait()
                scat[b][1].wait()

    return dispatch_kernel(xf, d0, d1)


def _ffn_body(te_ref, nt_ref, xs_ref, w1_ref, b1_ref, w2_ref, b2_ref, y_ref):
    i = pl.program_id(0)

    @pl.when(i < nt_ref[0])
    def _():
        x = xs_ref[...].astype(jnp.bfloat16)
        acc = jnp.zeros((T, D), jnp.float32)
        for f in range(DFF // FT):
            w1f = w1_ref[0, :, pl.ds(f * FT, FT)]
            h = jnp.dot(x, w1f, preferred_element_type=jnp.float32)
            h = h + b1_ref[0, 0, pl.ds(f * FT, FT)]
            h = h * (1.0 / (1.0 + jnp.exp(-h)))
            w2f = w2_ref[0, pl.ds(f * FT, FT), :]
            acc = acc + jnp.dot(
                h.astype(jnp.bfloat16), w2f, preferred_element_type=jnp.float32
            )
        y_ref[...] = acc + b2_ref[0]


def _ffn(xs, W1, b1, W2, b2, tile_expert, ntiles):
    grid_spec = pltpu.PrefetchScalarGridSpec(
        num_scalar_prefetch=2,
        grid=(NT,),
        in_specs=[
            pl.BlockSpec((T, D), lambda i, te, nt: (i, 0)),
            pl.BlockSpec((1, D, DFF), lambda i, te, nt: (te[i], 0, 0)),
            pl.BlockSpec((1, 1, DFF), lambda i, te, nt: (te[i], 0, 0)),
            pl.BlockSpec((1, DFF, D), lambda i, te, nt: (te[i], 0, 0)),
            pl.BlockSpec((1, 1, D), lambda i, te, nt: (te[i], 0, 0)),
        ],
        out_specs=pl.BlockSpec((T, D), lambda i, te, nt: (i, 0)),
    )
    return pl.pallas_call(
        _ffn_body,
        grid_spec=grid_spec,
        out_shape=jax.ShapeDtypeStruct((R, D), jnp.float32),
        compiler_params=pltpu.CompilerParams(
            dimension_semantics=("arbitrary",),
            vmem_limit_bytes=128 * 1024 * 1024,
        ),
    )(tile_expert, ntiles, xs, W1.astype(jnp.bfloat16), b1.reshape(E, 1, DFF),
      W2.astype(jnp.bfloat16), b2.reshape(E, 1, D))


_CCH = 16


def _combine(y, d0, d1, wb0, wb1):
    nit = TPW // (2 * _CCH)

    @functools.partial(
        pl.kernel,
        out_type=jax.ShapeDtypeStruct((N, D), jnp.float32),
        mesh=_sc_mesh(),
        scratch_types=[
            pltpu.VMEM((_CCH,), jnp.int32), pltpu.VMEM((_CCH,), jnp.int32),
            pltpu.VMEM((_CCH, 16), jnp.float32), pltpu.VMEM((_CCH, 16), jnp.float32),
            pltpu.VMEM((_CCH, D), jnp.float32), pltpu.VMEM((_CCH, D), jnp.float32),
            pltpu.VMEM((_CCH, D), jnp.float32),
            pltpu.VMEM((_CCH,), jnp.int32), pltpu.VMEM((_CCH,), jnp.int32),
            pltpu.VMEM((_CCH, 16), jnp.float32), pltpu.VMEM((_CCH, 16), jnp.float32),
            pltpu.VMEM((_CCH, D), jnp.float32), pltpu.VMEM((_CCH, D), jnp.float32),
            pltpu.VMEM((_CCH, D), jnp.float32),
            pltpu.SemaphoreType.DMA, pltpu.SemaphoreType.DMA,
            pltpu.SemaphoreType.DMA, pltpu.SemaphoreType.DMA,
            pltpu.SemaphoreType.DMA, pltpu.SemaphoreType.DMA,
        ],
    )
    def combine_kernel(y_hbm, d0_hbm, d1_hbm, w0_hbm, w1_hbm, out_hbm,
                       iA0, iA1, wA0, wA1, rA0, rA1, oA,
                       iB0, iB1, wB0, wB1, rB0, rB1, oB,
                       gsA0, gsA1, gsB0, gsB1, wsA, wsB):
        wid = lax.axis_index("s") * NC + lax.axis_index("c")
        wbase = wid * TPW

        def fetch(base, i0, i1, w0, w1, r0, r1, g0s, g1s):
            pltpu.sync_copy(d0_hbm.at[pl.ds(base, _CCH)], i0)
            pltpu.sync_copy(d1_hbm.at[pl.ds(base, _CCH)], i1)
            pltpu.sync_copy(w0_hbm.at[pl.ds(base, _CCH)], w0)
            pltpu.sync_copy(w1_hbm.at[pl.ds(base, _CCH)], w1)
            pltpu.async_copy(y_hbm.at[i0], r0, g0s)
            pltpu.async_copy(y_hbm.at[i1], r1, g1s)

        def compute(w0, w1, r0, r1, ob):
            def tbody(t, carry):
                wv0 = w0[t, :]
                wv1 = w1[t, :]
                for c in range(D // 16):
                    sl = pl.ds(c * 16, 16)
                    ob[t, sl] = wv0 * r0[t, sl] + wv1 * r1[t, sl]
                return carry

            lax.fori_loop(0, _CCH, tbody, 0)

        def gwait(r0, r1, g0s, g1s, i0, i1):
            pltpu.make_async_copy(y_hbm.at[i0], r0, g0s).wait()
            pltpu.make_async_copy(y_hbm.at[i1], r1, g1s).wait()

        fetch(wbase, iA0, iA1, wA0, wA1, rA0, rA1, gsA0, gsA1)
        fetch(wbase + _CCH, iB0, iB1, wB0, wB1, rB0, rB1, gsB0, gsB1)

        def mbody(m, carry):
            baseA = wbase + m * 2 * _CCH

            @pl.when(m > 0)
            def _():
                pltpu.make_async_copy(oA, out_hbm.at[pl.ds(baseA - 2 * _CCH, _CCH)], wsA).wait()
                pltpu.make_async_copy(oB, out_hbm.at[pl.ds(baseA - _CCH, _CCH)], wsB).wait()

            gwait(rA0, rA1, gsA0, gsA1, iA0, iA1)
            compute(wA0, wA1, rA0, rA1, oA)
            pltpu.async_copy(oA, out_hbm.at[pl.ds(baseA, _CCH)], wsA)
            gwait(rB0, rB1, gsB0, gsB1, iB0, iB1)
            compute(wB0, wB1, rB0, rB1, oB)
            pltpu.async_copy(oB, out_hbm.at[pl.ds(baseA + _CCH, _CCH)], wsB)

            @pl.when(m < nit - 1)
            def _():
                nbase = baseA + 2 * _CCH
                fetch(nbase, iA0, iA1, wA0, wA1, rA0, rA1, gsA0, gsA1)
                fetch(nbase + _CCH, iB0, iB1, wB0, wB1, rB0, rB1, gsB0, gsB1)

            return carry

        lax.fori_loop(0, nit, mbody, 0)
        last = wbase + (nit - 1) * 2 * _CCH
        pltpu.make_async_copy(oA, out_hbm.at[pl.ds(last, _CCH)], wsA).wait()
        pltpu.make_async_copy(oB, out_hbm.at[pl.ds(last + _CCH, _CCH)], wsB).wait()

    return combine_kernel(y, d0, d1, wb0, wb1)


def kernel(x, gate_W, gate_b, W1, b1, W2, b2):
    xf = x.reshape(N, D)
    wb0, wb1, idx = _gating(xf, gate_W, gate_b)
    tile_expert, ntiles, d0, d1 = _route_meta(idx)
    xs = _dispatch(xf, d0, d1)
    y = _ffn(xs, W1, b1, W2, b2, tile_expert, ntiles)
    out = _combine(y, d0, d1, wb0, wb1)
    return out.reshape(B, S, D)


# speedup vs baseline: 1.0460x; 1.0018x over previous
import functools

import jax
import jax.numpy as jnp
from jax import lax
from jax.experimental import pallas as pl
from jax.experimental.pallas import tpu as pltpu
from jax.experimental.pallas import tpu_sc as plsc

B, S, D, E, TOP_K = 4, 2048, 1024, 8, 2
DFF = 4 * D
N = B * S
P = N * TOP_K
T = 512
R = P + E * T
NT = R // T
FT = 1024

NC, NS = 2, 16
NW = NC * NS
TPW = N // NW


_GT = 512


def _gating_body(x_ref, w_ref, b_ref, w0_ref, w1_ref, idx_ref):
    x = x_ref[...]
    g = jnp.dot(x, w_ref[...], preferred_element_type=jnp.float32)
    g = (g + b_ref[...]) / (1.0 + 1e-06)
    iota = lax.broadcasted_iota(jnp.int32, g.shape, 1)
    m1 = jnp.max(g, axis=1, keepdims=True)
    a1 = jnp.min(jnp.where(g == m1, iota, E), axis=1, keepdims=True)
    g2 = jnp.where(iota == a1, -jnp.inf, g)
    m2 = jnp.max(g2, axis=1, keepdims=True)
    a2 = jnp.min(jnp.where(g2 == m2, iota, E), axis=1, keepdims=True)
    e = jnp.exp(m2 - m1)
    s = 1.0 + e
    w0_ref[...] = jnp.broadcast_to(1.0 / s, (_GT, 16))
    w1_ref[...] = jnp.broadcast_to(e / s, (_GT, 16))
    idx_ref[...] = jnp.concatenate([a1, a2], axis=1)


def _gating(xf, gate_W, gate_b):
    return pl.pallas_call(
        _gating_body,
        grid=(N // _GT,),
        in_specs=[
            pl.BlockSpec((_GT, D), lambda i: (i, 0)),
            pl.BlockSpec((D, E), lambda i: (0, 0)),
            pl.BlockSpec((1, E), lambda i: (0, 0)),
        ],
        out_specs=[
            pl.BlockSpec((_GT, 16), lambda i: (i, 0)),
            pl.BlockSpec((_GT, 16), lambda i: (i, 0)),
            pl.BlockSpec((_GT, TOP_K), lambda i: (i, 0)),
        ],
        out_shape=[
            jax.ShapeDtypeStruct((N, 16), jnp.float32),
            jax.ShapeDtypeStruct((N, 16), jnp.float32),
            jax.ShapeDtypeStruct((N, TOP_K), jnp.int32),
        ],
    )(xf, gate_W, gate_b.reshape(1, E))


_MG = 128
_NG = N // _MG


def _route_meta(idx):
    eids = jnp.arange(E, dtype=jnp.int32)
    A = (idx[:, 0][:, None] == eids).astype(jnp.float32)
    Bo = (idx[:, 1][:, None] == eids).astype(jnp.float32)
    tril_g = jnp.tril(jnp.ones((_NG, _NG), jnp.float32), k=-1)
    tril_m = jnp.tril(jnp.ones((_MG, _MG), jnp.float32))

    def excl_prefix(X):
        X3 = X.reshape(_NG, _MG, E)
        s = X3.sum(axis=1)
        gex = jnp.einsum("ij,je->ie", tril_g, s)
        incl = jnp.einsum("lk,gke->gle", tril_m, X3)
        return (incl - X3 + gex[:, None, :]).reshape(N, E), s.sum(axis=0)

    exA, cA = excl_prefix(A)
    exB, cB = excl_prefix(Bo)
    counts = (cA + cB).astype(jnp.int32)
    pc = ((counts + T - 1) // T) * T
    offs = jnp.concatenate([jnp.zeros(1, jnp.int32), jnp.cumsum(pc)]).astype(jnp.int32)
    rank0 = jnp.sum(A * (exA + exB), axis=1)
    rank1 = jnp.sum(Bo * (exA + A + exB), axis=1)
    offs_f = offs[:E].astype(jnp.float32)
    d0 = (rank0 + jnp.sum(A * offs_f, axis=1)).astype(jnp.int32)
    d1 = (rank1 + jnp.sum(Bo * offs_f, axis=1)).astype(jnp.int32)
    tile_starts = jnp.arange(NT, dtype=jnp.int32) * T
    tile_expert = jnp.clip(
        jnp.sum((tile_starts[:, None] >= offs[None, 1:E]).astype(jnp.int32), axis=1),
        0, E - 1,
    ).astype(jnp.int32)
    ntiles = (offs[E] // T).reshape(1)
    return tile_expert, ntiles, d0, d1


def _sc_mesh():
    return plsc.VectorSubcoreMesh(core_axis_name="c", subcore_axis_name="s")


_DCH = 32


def _dispatch(xf, d0, d1):
    nch = TPW // _DCH

    nbuf = 3

    @functools.partial(
        pl.kernel,
        out_type=jax.ShapeDtypeStruct((R, D), jnp.float32),
        mesh=_sc_mesh(),
        scratch_types=[
            pltpu.VMEM((_DCH,), jnp.int32), pltpu.VMEM((_DCH,), jnp.int32),
            pltpu.VMEM((_DCH,), jnp.int32), pltpu.VMEM((_DCH,), jnp.int32),
            pltpu.VMEM((_DCH,), jnp.int32), pltpu.VMEM((_DCH,), jnp.int32),
            pltpu.VMEM((_DCH, D), jnp.float32), pltpu.VMEM((_DCH, D), jnp.float32),
            pltpu.VMEM((_DCH, D), jnp.float32),
            pltpu.SemaphoreType.DMA, pltpu.SemaphoreType.DMA,
            pltpu.SemaphoreType.DMA,
            pltpu.SemaphoreType.DMA, pltpu.SemaphoreType.DMA,
            pltpu.SemaphoreType.DMA,
            pltpu.SemaphoreType.DMA, pltpu.SemaphoreType.DMA,
            pltpu.SemaphoreType.DMA,
        ],
    )
    def dispatch_kernel(x_hbm, d0_hbm, d1_hbm, xs_hbm,
                        i0a, i0b, i0c, i1a, i1b, i1c, ra, rb, rc,
                        la, lb, lc, s0a, s0b, s0c, s1a, s1b, s1c):
        wid = lax.axis_index("s") * NC + lax.axis_index("c")
        idx0 = [i0a, i0b, i0c]
        idx1 = [i1a, i1b, i1c]
        rows = [ra, rb, rc]
        lsem = [la, lb, lc]
        ssem0 = [s0a, s0b, s0c]
        ssem1 = [s1a, s1b, s1c]
        loads = [None] * nbuf
        scat = [[None, None] for _ in range(nbuf)]
        for j in range(nch + 1):
            if j < nch:
                b = j % nbuf
                base = wid * TPW + j * _DCH
                if scat[b][0] is not None:
                    scat[b][0].wait()
                    scat[b][1].wait()
                loads[b] = pltpu.async_copy(x_hbm.at[pl.ds(base, _DCH)], rows[b], lsem[b])
                pltpu.sync_copy(d0_hbm.at[pl.ds(base, _DCH)], idx0[b])
                pltpu.sync_copy(d1_hbm.at[pl.ds(base, _DCH)], idx1[b])
            if j >= 1:
                pb = (j - 1) % nbuf
                loads[pb].wait()
                scat[pb][0] = pltpu.async_copy(rows[pb], xs_hbm.at[idx0[pb]], ssem0[pb])
                scat[pb][1] = pltpu.async_copy(rows[pb], xs_hbm.at[idx1[pb]], ssem1[pb])
        for b in range(nbuf):
            if scat[b][0] is not None:
                scat[b][0].wait()
                scat[b][1].wait()

    return dispatch_kernel(xf, d0, d1)


def _ffn_body(te_ref, nt_ref, xs_ref, w1_ref, b1_ref, w2_ref, b2_ref, y_ref):
    i = pl.program_id(0)

    @pl.when(i < nt_ref[0])
    def _():
        x = xs_ref[...].astype(jnp.bfloat16)
        acc = jnp.zeros((T, D), jnp.float32)
        for f in range(DFF // FT):
            w1f = w1_ref[0, :, pl.ds(f * FT, FT)]
            h = jnp.dot(x, w1f, preferred_element_type=jnp.float32)
            h = h + b1_ref[0, 0, pl.ds(f * FT, FT)]
            h = h * (1.0 / (1.0 + jnp.exp(-h)))
            w2f = w2_ref[0, pl.ds(f * FT, FT), :]
            acc = acc + jnp.dot(
                h.astype(jnp.bfloat16), w2f, preferred_element_type=jnp.float32
            )
        y_ref[...] = acc + b2_ref[0]


def _ffn(xs, W1, b1, W2, b2, tile_expert, ntiles):
    grid_spec = pltpu.PrefetchScalarGridSpec(
        num_scalar_prefetch=2,
        grid=(NT,),
        in_specs=[
            pl.BlockSpec((T, D), lambda i, te, nt: (i, 0)),
            pl.BlockSpec((1, D, DFF), lambda i, te, nt: (te[i], 0, 0)),
            pl.BlockSpec((1, 1, DFF), lambda i, te, nt: (te[i], 0, 0)),
            pl.BlockSpec((1, DFF, D), lambda i, te, nt: (te[i], 0, 0)),
            pl.BlockSpec((1, 1, D), lambda i, te, nt: (te[i], 0, 0)),
        ],
        out_specs=pl.BlockSpec((T, D), lambda i, te, nt: (i, 0)),
    )
    return pl.pallas_call(
        _ffn_body,
        grid_spec=grid_spec,
        out_shape=jax.ShapeDtypeStruct((R, D), jnp.float32),
        compiler_params=pltpu.CompilerParams(
            dimension_semantics=("arbitrary",),
            vmem_limit_bytes=128 * 1024 * 1024,
        ),
    )(tile_expert, ntiles, xs, W1.astype(jnp.bfloat16), b1.reshape(E, 1, DFF),
      W2.astype(jnp.bfloat16), b2.reshape(E, 1, D))


_CCH = 16


def _combine(y, d0, d1, wb0, wb1):
    nit = TPW // (2 * _CCH)

    @functools.partial(
        pl.kernel,
        out_type=jax.ShapeDtypeStruct((N, D), jnp.float32),
        mesh=_sc_mesh(),
        scratch_types=[
            pltpu.VMEM((_CCH,), jnp.int32), pltpu.VMEM((_CCH,), jnp.int32),
            pltpu.VMEM((_CCH, 16), jnp.float32), pltpu.VMEM((_CCH, 16), jnp.float32),
            pltpu.VMEM((_CCH, D), jnp.float32), pltpu.VMEM((_CCH, D), jnp.float32),
            pltpu.VMEM((_CCH, D), jnp.float32),
            pltpu.VMEM((_CCH,), jnp.int32), pltpu.VMEM((_CCH,), jnp.int32),
            pltpu.VMEM((_CCH, 16), jnp.float32), pltpu.VMEM((_CCH, 16), jnp.float32),
            pltpu.VMEM((_CCH, D), jnp.float32), pltpu.VMEM((_CCH, D), jnp.float32),
            pltpu.VMEM((_CCH, D), jnp.float32),
            pltpu.SemaphoreType.DMA, pltpu.SemaphoreType.DMA,
            pltpu.SemaphoreType.DMA, pltpu.SemaphoreType.DMA,
            pltpu.SemaphoreType.DMA, pltpu.SemaphoreType.DMA,
        ],
    )
    def combine_kernel(y_hbm, d0_hbm, d1_hbm, w0_hbm, w1_hbm, out_hbm,
                       iA0, iA1, wA0, wA1, rA0, rA1, oA,
                       iB0, iB1, wB0, wB1, rB0, rB1, oB,
                       gsA0, gsA1, gsB0, gsB1, wsA, wsB):
        wid = lax.axis_index("s") * NC + lax.axis_index("c")
        wbase = wid * TPW

        def fetch(base, i0, i1, w0, w1, r0, r1, g0s, g1s):
            pltpu.sync_copy(d0_hbm.at[pl.ds(base, _CCH)], i0)
            pltpu.sync_copy(d1_hbm.at[pl.ds(base, _CCH)], i1)
            pltpu.sync_copy(w0_hbm.at[pl.ds(base, _CCH)], w0)
            pltpu.sync_copy(w1_hbm.at[pl.ds(base, _CCH)], w1)
            pltpu.async_copy(y_hbm.at[i0], r0, g0s)
            pltpu.async_copy(y_hbm.at[i1], r1, g1s)

        def compute(w0, w1, r0, r1, ob):
            def tbody(t, carry):
                wv0 = w0[t, :]
                wv1 = w1[t, :]
                for c in range(D // 16):
                    sl = pl.ds(c * 16, 16)
                    ob[t, sl] = wv0 * r0[t, sl] + wv1 * r1[t, sl]
                return carry

            lax.fori_loop(0, _CCH, tbody, 0)

        def gwait(r0, r1, g0s, g1s, i0, i1):
            pltpu.make_async_copy(y_hbm.at[i0], r0, g0s).wait()
            pltpu.make_async_copy(y_hbm.at[i1], r1, g1s).wait()

        fetch(wbase, iA0, iA1, wA0, wA1, rA0, rA1, gsA0, gsA1)
        fetch(wbase + _CCH, iB0, iB1, wB0, wB1, rB0, rB1, gsB0, gsB1)

        def mbody(m, carry):
            baseA = wbase + m * 2 * _CCH

            @pl.when(m > 0)
            def _():
                pltpu.make_async_copy(oA, out_hbm.at[pl.ds(baseA - 2 * _CCH, _CCH)], wsA).wait()
                pltpu.make_async_copy(oB, out_hbm.at[pl.ds(baseA - _CCH, _CCH)], wsB).wait()

            gwait(rA0, rA1, gsA0, gsA1, iA0, iA1)
            compute(wA0, wA1, rA0, rA1, oA)
            pltpu.async_copy(oA, out_hbm.at[pl.ds(baseA, _CCH)], wsA)
            gwait(rB0, rB1, gsB0, gsB1, iB0, iB1)
            compute(wB0, wB1, rB0, rB1, oB)
            pltpu.async_copy(oB, out_hbm.at[pl.ds(baseA + _CCH, _CCH)], wsB)

            @pl.when(m < nit - 1)
            def _():
                nbase = baseA + 2 * _CCH
                fetch(nbase, iA0, iA1, wA0, wA1, rA0, rA1, gsA0, gsA1)
                fetch(nbase + _CCH, iB0, iB1, wB0, wB1, rB0, rB1, gsB0, gsB1)

            return carry

        lax.fori_loop(0, nit, mbody, 0)
        last = wbase + (nit - 1) * 2 * _CCH
        pltpu.make_async_copy(oA, out_hbm.at[pl.ds(last, _CCH)], wsA).wait()
        pltpu.make_async_copy(oB, out_hbm.at[pl.ds(last + _CCH, _CCH)], wsB).wait()

    return combine_kernel(y, d0, d1, wb0, wb1)


def kernel(x, gate_W, gate_b, W1, b1, W2, b2):
    xf = x.reshape(N, D)
    wb0, wb1, idx = _gating(xf, gate_W, gate_b)
    tile_expert, ntiles, d0, d1 = _route_meta(idx)
    xs = _dispatch(xf, d0, d1)
    y = _ffn(xs, W1, b1, W2, b2, tile_expert, ntiles)
    out = _combine(y, d0, d1, wb0, wb1)
    return out.reshape(B, S, D)
